# baseline (speedup 1.0000x reference)
import jax
import jax.numpy as jnp
from jax import lax
from jax.experimental import pallas as pl
from jax.experimental.pallas import tpu as pltpu
from jax.experimental.pallas import tpu_sc as plsc

_NC = 2
_NS = 16
_CHUNK = 128
_GRP = 8
_RB = 8
_LA = 4


def _fill_rows(ref, n_rows, n_cols, value):
  v = jnp.full((16,), value, jnp.float32)

  def body(i, carry):
    for j in range(n_cols // 16):
      ref[i, pl.ds(j * 16, 16)] = v
    return carry

  lax.fori_loop(0, n_rows, body, 0)


def _npad(n_nodes):
  return (-(-n_nodes // (_NS * 8)) * 8) * _NS


def _idx_rows(n_edges):
  n_chunks = -(-n_edges // _CHUNK)
  return -(-n_chunks // _GRP) * _GRP


def _make_seg_sum(n_nodes, dim_half, n_edges_pad, with_deg):
  assert n_edges_pad % (_CHUNK * _GRP) == 0
  idx_rows = n_edges_pad // _CHUNK
  n_groups = idx_rows // _GRP
  n_iters = -(-n_groups // _NS)
  npad = _npad(n_nodes)
  rps = npad // _NS
  assert rps % _CHUNK == 0 or rps < _CHUNK or True

  mesh = plsc.VectorSubcoreMesh(
      core_axis_name="c", subcore_axis_name="s",
      num_cores=_NC, num_subcores=_NS)

  outs = [jax.ShapeDtypeStruct((_NC * npad, dim_half), jnp.float32)]
  scratch = [
      pltpu.VMEM((_GRP, _CHUNK), jnp.int32),
      pltpu.VMEM((_GRP, _CHUNK), jnp.int32),
  ] + [pltpu.VMEM((_CHUNK, dim_half), jnp.float32) for _ in range(_RB)
  ] + [pltpu.VMEM_SHARED((npad, dim_half), jnp.float32)
  ] + [pltpu.SemaphoreType.DMA for _ in range(2 * _RB)]
  if with_deg:
    outs.append(jax.ShapeDtypeStruct((_NC * npad, 16), jnp.float32))
    scratch += [
        pltpu.VMEM((_CHUNK, 16), jnp.float32),
        pltpu.VMEM((_CHUNK, 16), jnp.float32),
        pltpu.VMEM_SHARED((npad, 16), jnp.float32),
        pltpu.SemaphoreType.DMA,
    ]

  def body(x_hbm, srcr_hbm, dstr_hbm, acc_out, *rest):
    if with_deg:
      deg_out = rest[0]
      rest = rest[1:]
    src_v, dst_v = rest[0], rest[1]
    bufs = rest[2:2 + _RB]
    acc_sh = rest[2 + _RB]
    gsems = rest[3 + _RB:3 + 2 * _RB]
    ssems = rest[3 + 2 * _RB:3 + 3 * _RB]
    if with_deg:
      ones_v, dstage_v, deg_sh, dsem = rest[3 + 3 * _RB:]

    cid = lax.axis_index("c")
    sid = lax.axis_index("s")
    row0 = sid * rps

    _fill_rows(bufs[0], _CHUNK, dim_half, 0.0)
    zds = []
    done = 0
    while done < rps:
      cnt = min(_CHUNK, rps - done)
      zds.append(pltpu.async_copy(bufs[0].at[pl.ds(0, cnt)],
                                  acc_sh.at[pl.ds(row0 + done, cnt)],
                                  gsems[0]))
      done += cnt
    if with_deg:
      _fill_rows(ones_v, _CHUNK, 16, 1.0)
      _fill_rows(dstage_v, _CHUNK, 16, 0.0)
      done = 0
      while done < rps:
        cnt = min(_CHUNK, rps - done)
        zds.append(pltpu.async_copy(dstage_v.at[pl.ds(0, cnt)],
                                    deg_sh.at[pl.ds(row0 + done, cnt)],
                                    gsems[1]))
        done += cnt
    for d in zds:
      d.wait()
    plsc.subcore_barrier()

    def step(jg, carry):
      g = sid + _NS * jg

      @pl.when(g < n_groups)
      def _():
        pltpu.sync_copy(srcr_hbm.at[pl.ds(g * _GRP, _GRP)], src_v)
        pltpu.sync_copy(dstr_hbm.at[pl.ds(g * _GRP, _GRP)], dst_v)

        src_off = cid * n_nodes

        def xform(r):
          for j in range(_CHUNK // 16):
            sl = pl.ds(j * 16, 16)
            src_v[r, sl] = src_v[r, sl] + src_off

        gd = {}
        sd = {}
        for k in range(min(_LA, _GRP)):
          xform(k)
          gd[k] = pltpu.async_copy(x_hbm.at[src_v.at[k]], bufs[k % _RB],
                                   gsems[k % _RB])
        for k in range(_GRP):
          gd[k].wait()
          sd[k] = pltpu.async_copy(bufs[k % _RB], acc_sh.at[dst_v.at[k]],
                                   ssems[k % _RB], add=True)
          c = k + _LA
          if c < _GRP:
            if c - _RB >= 0:
              sd[c - _RB].wait()
            xform(c)
            gd[c] = pltpu.async_copy(x_hbm.at[src_v.at[c]], bufs[c % _RB],
                                     gsems[c % _RB])
        for k in range(max(0, _GRP - _RB), _GRP):
          sd[k].wait()
        if with_deg:
          @pl.when((sid % 2) == cid)
          def _():
            dsd = [pltpu.async_copy(ones_v, deg_sh.at[dst_v.at[k]],
                                    dsem, add=True)
                   for k in range(_GRP)]
            for d in dsd:
              d.wait()

      return carry

    lax.fori_loop(0, n_iters, step, 0)
    plsc.subcore_barrier()

    out_row0 = cid * npad + row0
    spans = []
    done = 0
    while done < rps:
      cnt = min(_CHUNK, rps - done)
      spans.append((done, cnt))
      done += cnt
    ins = {}
    outs_d = {}
    ins[0] = pltpu.async_copy(acc_sh.at[pl.ds(row0 + spans[0][0],
                                              spans[0][1])],
                              bufs[0].at[pl.ds(0, spans[0][1])], gsems[0])
    for i, (off, cnt) in enumerate(spans):
      if i + 1 < len(spans):
        if i - 1 >= 0:
          outs_d[i - 1].wait()
        off2, cnt2 = spans[i + 1]
        ins[i + 1] = pltpu.async_copy(
            acc_sh.at[pl.ds(row0 + off2, cnt2)],
            bufs[(i + 1) % 2].at[pl.ds(0, cnt2)], gsems[(i + 1) % 2])
      ins[i].wait()
      outs_d[i] = pltpu.async_copy(
          bufs[i % 2].at[pl.ds(0, cnt)],
          acc_out.at[pl.ds(out_row0 + off, cnt)], ssems[i % 2])
    for i in range(max(0, len(spans) - 2), len(spans)):
      outs_d[i].wait()
    if with_deg:
      deg_row0 = cid * npad + row0
      done = 0
      while done < rps:
        cnt = min(_CHUNK, rps - done)
        pltpu.sync_copy(deg_sh.at[pl.ds(row0 + done, cnt)],
                        dstage_v.at[pl.ds(0, cnt)])
        pltpu.sync_copy(dstage_v.at[pl.ds(0, cnt)],
                        deg_out.at[pl.ds(deg_row0 + done, cnt)])
        done += cnt

  return pl.kernel(body, out_type=tuple(outs), mesh=mesh,
                   scratch_types=scratch,
                   compiler_params=pltpu.CompilerParams(
                       use_tc_tiling_on_sc=False))


def _layer0_and_proj(h, p, d, W0, b0, W1, Wp):
  n, in_dim = h.shape
  npad = _npad(n)
  blk = 1000
  hd = in_dim // 2
  hid = W0.shape[1]
  nc = W1.shape[1]

  def body(h_ref, p_ref, d_ref, w0_ref, b0_ref, w1_ref, wp_ref,
           z_ref, pr_ref):
    deg = jnp.maximum(d_ref[0, :, 0:1] + d_ref[1, :, 0:1], 1.0)
    neigh = jnp.concatenate([p_ref[0], p_ref[1]], axis=1) / deg
    x0 = h_ref[...] + neigh
    h0 = jnp.maximum(
        jnp.dot(x0, w0_ref[...], preferred_element_type=jnp.float32)
        + b0_ref[...], 0.0)
    z = jnp.dot(h0, w1_ref[...], preferred_element_type=jnp.float32)
    z_ref[0] = z[:, :nc // 2]
    z_ref[1] = z[:, nc // 2:]
    pr_ref[...] = jnp.dot(h0, wp_ref[...], preferred_element_type=jnp.float32)

  return pl.pallas_call(
      body,
      grid=(n // blk,),
      in_specs=[
          pl.BlockSpec((blk, in_dim), lambda i: (i, 0)),
          pl.BlockSpec((2, blk, hd), lambda i: (0, i, 0)),
          pl.BlockSpec((2, blk, 16), lambda i: (0, i, 0)),
          pl.BlockSpec((in_dim, hid), lambda i: (0, 0)),
          pl.BlockSpec((1, hid), lambda i: (0, 0)),
          pl.BlockSpec((hid, nc), lambda i: (0, 0)),
          pl.BlockSpec((hid, nc), lambda i: (0, 0)),
      ],
      out_specs=[
          pl.BlockSpec((2, blk, nc // 2), lambda i: (0, i, 0)),
          pl.BlockSpec((blk, nc), lambda i: (i, 0)),
      ],
      out_shape=[
          jax.ShapeDtypeStruct((2, n, nc // 2), jnp.float32),
          jax.ShapeDtypeStruct((n, nc), jnp.float32),
      ],
  )(h, p.reshape(2, npad, hd), d.reshape(2, npad, 16), W0,
    b0.reshape(1, -1), W1, Wp)


def _layer1_combine(z_pair, q, d, b1, proj):
  n = proj.shape[0]
  nc = proj.shape[1]
  npad = _npad(n)
  blk = 1000

  def body(z_ref, q_ref, d_ref, b1_ref, pr_ref, o_ref):
    deg = jnp.maximum(d_ref[0, :, 0:1] + d_ref[1, :, 0:1], 1.0)
    z = jnp.concatenate([z_ref[0], z_ref[1]], axis=1)
    neigh = jnp.concatenate([q_ref[0], q_ref[1]], axis=1) / deg
    h1 = jnp.maximum(z + neigh + b1_ref[...], 0.0)
    o_ref[...] = (pr_ref[...] + h1) * 0.5

  return pl.pallas_call(
      body,
      grid=(n // blk,),
      in_specs=[
          pl.BlockSpec((2, blk, nc // 2), lambda i: (0, i, 0)),
          pl.BlockSpec((2, blk, nc // 2), lambda i: (0, i, 0)),
          pl.BlockSpec((2, blk, 16), lambda i: (0, i, 0)),
          pl.BlockSpec((1, nc), lambda i: (0, 0)),
          pl.BlockSpec((blk, nc), lambda i: (i, 0)),
      ],
      out_specs=pl.BlockSpec((blk, nc), lambda i: (i, 0)),
      out_shape=jax.ShapeDtypeStruct((n, nc), jnp.float32),
  )(z_pair, q.reshape(2, npad, nc // 2), d.reshape(2, npad, 16),
    b1.reshape(1, -1), proj)


def kernel(h, edge_index, W0, b0, W1, b1, Wp):
  n, in_dim = h.shape
  e = edge_index.shape[1]
  nc = W1.shape[1]
  npad = _npad(n)
  src = edge_index[0]
  dst = edge_index[1]

  idx_rows = _idx_rows(e)
  e_pad = idx_rows * _CHUNK
  pad = e_pad - e
  src_p = jnp.concatenate([src, jnp.zeros((pad,), jnp.int32)])
  dst_p = jnp.concatenate(
      [dst, n + (jnp.arange(pad, dtype=jnp.int32) % (npad - n))])
  srcr = src_p.reshape(idx_rows, _CHUNK)
  dstr = dst_p.reshape(idx_rows, _CHUNK)

  h_pair = jnp.concatenate([h[:, :in_dim // 2], h[:, in_dim // 2:]], axis=0)

  seg0 = _make_seg_sum(n, in_dim // 2, e_pad, with_deg=True)
  p_flat, d_flat = seg0(h_pair, srcr, dstr)

  z_pair, proj = _layer0_and_proj(h, p_flat, d_flat, W0, b0, W1, Wp)

  seg1 = _make_seg_sum(n, nc // 2, e_pad, with_deg=False)
  res = seg1(z_pair.reshape(2 * n, nc // 2), srcr, dstr)
  q_flat = res[0] if isinstance(res, (tuple, list)) else res

  return _layer1_combine(z_pair, q_flat, d_flat, b1, proj)


# speedup vs baseline: 1.0376x; 1.0376x over previous
import jax
import jax.numpy as jnp
from jax import lax
from jax.experimental import pallas as pl
from jax.experimental.pallas import tpu as pltpu
from jax.experimental.pallas import tpu_sc as plsc

_NC = 2
_NS = 16
_CHUNK = 128
_GRP = 16
_RB = 8
_LA = 6


def _fill_rows(ref, n_rows, n_cols, value):
  v = jnp.full((16,), value, jnp.float32)

  def body(i, carry):
    for j in range(n_cols // 16):
      ref[i, pl.ds(j * 16, 16)] = v
    return carry

  lax.fori_loop(0, n_rows, body, 0)


def _npad(n_nodes):
  return (-(-n_nodes // (_NS * 8)) * 8) * _NS


def _idx_rows(n_edges):
  n_chunks = -(-n_edges // _CHUNK)
  return -(-n_chunks // _GRP) * _GRP


def _make_seg_sum(n_nodes, dim_half, n_edges_pad, with_deg):
  assert n_edges_pad % (_CHUNK * _GRP) == 0
  idx_rows = n_edges_pad // _CHUNK
  n_groups = idx_rows // _GRP
  n_iters = -(-n_groups // _NS)
  npad = _npad(n_nodes)
  rps = npad // _NS
  assert rps % _CHUNK == 0 or rps < _CHUNK or True

  mesh = plsc.VectorSubcoreMesh(
      core_axis_name="c", subcore_axis_name="s",
      num_cores=_NC, num_subcores=_NS)

  outs = [jax.ShapeDtypeStruct((_NC * npad, dim_half), jnp.float32)]
  scratch = [
      pltpu.VMEM((_GRP, _CHUNK), jnp.int32),
      pltpu.VMEM((_GRP, _CHUNK), jnp.int32),
  ] + [pltpu.VMEM((_CHUNK, dim_half), jnp.float32) for _ in range(_RB)
  ] + [pltpu.VMEM_SHARED((npad, dim_half), jnp.float32)
  ] + [pltpu.SemaphoreType.DMA for _ in range(2 * _RB)]
  if with_deg:
    outs.append(jax.ShapeDtypeStruct((_NC * npad, 16), jnp.float32))
    scratch += [
        pltpu.VMEM((_CHUNK, 16), jnp.float32),
        pltpu.VMEM((_CHUNK, 16), jnp.float32),
        pltpu.VMEM_SHARED((npad, 16), jnp.float32),
        pltpu.SemaphoreType.DMA,
    ]

  def body(x_hbm, srcr_hbm, dstr_hbm, acc_out, *rest):
    if with_deg:
      deg_out = rest[0]
      rest = rest[1:]
    src_v, dst_v = rest[0], rest[1]
    bufs = rest[2:2 + _RB]
    acc_sh = rest[2 + _RB]
    gsems = rest[3 + _RB:3 + 2 * _RB]
    ssems = rest[3 + 2 * _RB:3 + 3 * _RB]
    if with_deg:
      ones_v, dstage_v, deg_sh, dsem = rest[3 + 3 * _RB:]

    cid = lax.axis_index("c")
    sid = lax.axis_index("s")
    row0 = sid * rps

    _fill_rows(bufs[0], _CHUNK, dim_half, 0.0)
    zds = []
    done = 0
    while done < rps:
      cnt = min(_CHUNK, rps - done)
      zds.append(pltpu.async_copy(bufs[0].at[pl.ds(0, cnt)],
                                  acc_sh.at[pl.ds(row0 + done, cnt)],
                                  gsems[0]))
      done += cnt
    if with_deg:
      _fill_rows(ones_v, _CHUNK, 16, 1.0)
      _fill_rows(dstage_v, _CHUNK, 16, 0.0)
      done = 0
      while done < rps:
        cnt = min(_CHUNK, rps - done)
        zds.append(pltpu.async_copy(dstage_v.at[pl.ds(0, cnt)],
                                    deg_sh.at[pl.ds(row0 + done, cnt)],
                                    gsems[1]))
        done += cnt
    for d in zds:
      d.wait()
    plsc.subcore_barrier()

    def step(jg, carry):
      g = sid + _NS * jg

      @pl.when(g < n_groups)
      def _():
        pltpu.sync_copy(srcr_hbm.at[pl.ds(g * _GRP, _GRP)], src_v)
        pltpu.sync_copy(dstr_hbm.at[pl.ds(g * _GRP, _GRP)], dst_v)

        src_off = cid * n_nodes

        def xform(r):
          for j in range(_CHUNK // 16):
            sl = pl.ds(j * 16, 16)
            src_v[r, sl] = src_v[r, sl] + src_off

        gd = {}
        sd = {}
        for k in range(min(_LA, _GRP)):
          xform(k)
          gd[k] = pltpu.async_copy(x_hbm.at[src_v.at[k]], bufs[k % _RB],
                                   gsems[k % _RB])
        for k in range(_GRP):
          gd[k].wait()
          sd[k] = pltpu.async_copy(bufs[k % _RB], acc_sh.at[dst_v.at[k]],
                                   ssems[k % _RB], add=True)
          c = k + _LA
          if c < _GRP:
            if c - _RB >= 0:
              sd[c - _RB].wait()
            xform(c)
            gd[c] = pltpu.async_copy(x_hbm.at[src_v.at[c]], bufs[c % _RB],
                                     gsems[c % _RB])
        for k in range(max(0, _GRP - _RB), _GRP):
          sd[k].wait()
        if with_deg:
          @pl.when((sid % 2) == cid)
          def _():
            dsd = [pltpu.async_copy(ones_v, deg_sh.at[dst_v.at[k]],
                                    dsem, add=True)
                   for k in range(_GRP)]
            for d in dsd:
              d.wait()

      return carry

    lax.fori_loop(0, n_iters, step, 0)
    plsc.subcore_barrier()

    out_row0 = cid * npad + row0
    spans = []
    done = 0
    while done < rps:
      cnt = min(_CHUNK, rps - done)
      spans.append((done, cnt))
      done += cnt
    ins = {}
    outs_d = {}
    ins[0] = pltpu.async_copy(acc_sh.at[pl.ds(row0 + spans[0][0],
                                              spans[0][1])],
                              bufs[0].at[pl.ds(0, spans[0][1])], gsems[0])
    for i, (off, cnt) in enumerate(spans):
      if i + 1 < len(spans):
        if i - 1 >= 0:
          outs_d[i - 1].wait()
        off2, cnt2 = spans[i + 1]
        ins[i + 1] = pltpu.async_copy(
            acc_sh.at[pl.ds(row0 + off2, cnt2)],
            bufs[(i + 1) % 2].at[pl.ds(0, cnt2)], gsems[(i + 1) % 2])
      ins[i].wait()
      outs_d[i] = pltpu.async_copy(
          bufs[i % 2].at[pl.ds(0, cnt)],
          acc_out.at[pl.ds(out_row0 + off, cnt)], ssems[i % 2])
    for i in range(max(0, len(spans) - 2), len(spans)):
      outs_d[i].wait()
    if with_deg:
      deg_row0 = cid * npad + row0
      done = 0
      while done < rps:
        cnt = min(_CHUNK, rps - done)
        pltpu.sync_copy(deg_sh.at[pl.ds(row0 + done, cnt)],
                        dstage_v.at[pl.ds(0, cnt)])
        pltpu.sync_copy(dstage_v.at[pl.ds(0, cnt)],
                        deg_out.at[pl.ds(deg_row0 + done, cnt)])
        done += cnt

  return pl.kernel(body, out_type=tuple(outs), mesh=mesh,
                   scratch_types=scratch,
                   compiler_params=pltpu.CompilerParams(
                       use_tc_tiling_on_sc=False))


def _layer0_and_proj(h, p, d, W0, b0, W1, Wp):
  n, in_dim = h.shape
  npad = _npad(n)
  blk = 1000
  hd = in_dim // 2
  hid = W0.shape[1]
  nc = W1.shape[1]

  def body(h_ref, p_ref, d_ref, w0_ref, b0_ref, w1_ref, wp_ref,
           z_ref, pr_ref):
    deg = jnp.maximum(d_ref[0, :, 0:1] + d_ref[1, :, 0:1], 1.0)
    neigh = jnp.concatenate([p_ref[0], p_ref[1]], axis=1) / deg
    x0 = h_ref[...] + neigh
    h0 = jnp.maximum(
        jnp.dot(x0, w0_ref[...], preferred_element_type=jnp.float32)
        + b0_ref[...], 0.0)
    z = jnp.dot(h0, w1_ref[...], preferred_element_type=jnp.float32)
    z_ref[0] = z[:, :nc // 2]
    z_ref[1] = z[:, nc // 2:]
    pr_ref[...] = jnp.dot(h0, wp_ref[...], preferred_element_type=jnp.float32)

  return pl.pallas_call(
      body,
      grid=(n // blk,),
      in_specs=[
          pl.BlockSpec((blk, in_dim), lambda i: (i, 0)),
          pl.BlockSpec((2, blk, hd), lambda i: (0, i, 0)),
          pl.BlockSpec((2, blk, 16), lambda i: (0, i, 0)),
          pl.BlockSpec((in_dim, hid), lambda i: (0, 0)),
          pl.BlockSpec((1, hid), lambda i: (0, 0)),
          pl.BlockSpec((hid, nc), lambda i: (0, 0)),
          pl.BlockSpec((hid, nc), lambda i: (0, 0)),
      ],
      out_specs=[
          pl.BlockSpec((2, blk, nc // 2), lambda i: (0, i, 0)),
          pl.BlockSpec((blk, nc), lambda i: (i, 0)),
      ],
      out_shape=[
          jax.ShapeDtypeStruct((2, n, nc // 2), jnp.float32),
          jax.ShapeDtypeStruct((n, nc), jnp.float32),
      ],
  )(h, p.reshape(2, npad, hd), d.reshape(2, npad, 16), W0,
    b0.reshape(1, -1), W1, Wp)


def _layer1_combine(z_pair, q, d, b1, proj):
  n = proj.shape[0]
  nc = proj.shape[1]
  npad = _npad(n)
  blk = 1000

  def body(z_ref, q_ref, d_ref, b1_ref, pr_ref, o_ref):
    deg = jnp.maximum(d_ref[0, :, 0:1] + d_ref[1, :, 0:1], 1.0)
    z = jnp.concatenate([z_ref[0], z_ref[1]], axis=1)
    neigh = jnp.concatenate([q_ref[0], q_ref[1]], axis=1) / deg
    h1 = jnp.maximum(z + neigh + b1_ref[...], 0.0)
    o_ref[...] = (pr_ref[...] + h1) * 0.5

  return pl.pallas_call(
      body,
      grid=(n // blk,),
      in_specs=[
          pl.BlockSpec((2, blk, nc // 2), lambda i: (0, i, 0)),
          pl.BlockSpec((2, blk, nc // 2), lambda i: (0, i, 0)),
          pl.BlockSpec((2, blk, 16), lambda i: (0, i, 0)),
          pl.BlockSpec((1, nc), lambda i: (0, 0)),
          pl.BlockSpec((blk, nc), lambda i: (i, 0)),
      ],
      out_specs=pl.BlockSpec((blk, nc), lambda i: (i, 0)),
      out_shape=jax.ShapeDtypeStruct((n, nc), jnp.float32),
  )(z_pair, q.reshape(2, npad, nc // 2), d.reshape(2, npad, 16),
    b1.reshape(1, -1), proj)


def kernel(h, edge_index, W0, b0, W1, b1, Wp):
  n, in_dim = h.shape
  e = edge_index.shape[1]
  nc = W1.shape[1]
  npad = _npad(n)
  src = edge_index[0]
  dst = edge_index[1]

  idx_rows = _idx_rows(e)
  e_pad = idx_rows * _CHUNK
  pad = e_pad - e
  src_p = jnp.concatenate([src, jnp.zeros((pad,), jnp.int32)])
  dst_p = jnp.concatenate(
      [dst, n + (jnp.arange(pad, dtype=jnp.int32) % (npad - n))])
  srcr = src_p.reshape(idx_rows, _CHUNK)
  dstr = dst_p.reshape(idx_rows, _CHUNK)

  h_pair = jnp.concatenate([h[:, :in_dim // 2], h[:, in_dim // 2:]], axis=0)

  seg0 = _make_seg_sum(n, in_dim // 2, e_pad, with_deg=True)
  p_flat, d_flat = seg0(h_pair, srcr, dstr)

  z_pair, proj = _layer0_and_proj(h, p_flat, d_flat, W0, b0, W1, Wp)

  seg1 = _make_seg_sum(n, nc // 2, e_pad, with_deg=False)
  res = seg1(z_pair.reshape(2 * n, nc // 2), srcr, dstr)
  q_flat = res[0] if isinstance(res, (tuple, list)) else res

  return _layer1_combine(z_pair, q_flat, d_flat, b1, proj)
